# Initial kernel scaffold; baseline (speedup 1.0000x reference)
import jax
import jax.numpy as jnp
from jax.experimental import pallas as pl


def kernel(context_ids, center_ids, labels, context_table, center_table):
    raise NotImplementedError("write your pallas kernel here")


# speedup vs baseline: 6.1646x; 6.1646x over previous
import functools

import jax
import jax.numpy as jnp
from jax import lax
from jax.experimental import pallas as pl
from jax.experimental.pallas import tpu as pltpu
from jax.experimental.pallas import tpu_sc as plsc

VOCAB = 100000
EMBED = 64
BATCH = 16384
CTX = 20

NUM_CORES = 2
NUM_SUBCORES = 16
NW = NUM_CORES * NUM_SUBCORES
PER_W = BATCH // NW
CH = 64
N_CHUNKS = PER_W // CH
LANES = 16
NV = EMBED // LANES


def _sc_scores(ids_flat, center_ids, context_table, center_table):
  mesh = plsc.VectorSubcoreMesh(core_axis_name="c", subcore_axis_name="s")

  @functools.partial(
      pl.kernel,
      out_type=jax.ShapeDtypeStruct((BATCH,), jnp.float32),
      mesh=mesh,
      compiler_params=pltpu.CompilerParams(needs_layout_passes=False,
                                           use_tc_tiling_on_sc=False),
      scratch_types=[
          pltpu.VMEM((CH * CTX,), jnp.int32),
          pltpu.VMEM((CH,), jnp.int32),
          pltpu.VMEM((CH * CTX, EMBED), jnp.float32),
          pltpu.VMEM((CH, EMBED), jnp.float32),
          pltpu.VMEM((EMBED,), jnp.float32),
          pltpu.VMEM((CH,), jnp.float32),
          pltpu.SemaphoreType.DMA,
          pltpu.SemaphoreType.DMA,
      ],
  )
  def kern(ids_hbm, cids_hbm, ctab_hbm, gtab_hbm, out_hbm,
           idx_v, cidx_v, rows_v, crows_v, row0_v, sc_v, sem1, sem2):
    wid = lax.axis_index("s") * NUM_CORES + lax.axis_index("c")
    wbase = wid * PER_W
    lanes = lax.iota(jnp.int32, LANES)
    ones = jnp.ones((LANES,), jnp.int32)
    zeros = jnp.zeros((LANES,), jnp.int32)

    pltpu.sync_copy(ctab_hbm.at[0], row0_v)
    row0 = tuple(row0_v[pl.ds(v * LANES, LANES)] for v in range(NV))

    for c in range(N_CHUNKS):
      ebase = wbase + c * CH
      pltpu.sync_copy(ids_hbm.at[pl.ds(ebase * CTX, CH * CTX)], idx_v)
      pltpu.sync_copy(cids_hbm.at[pl.ds(ebase, CH)], cidx_v)
      cp1 = pltpu.async_copy(ctab_hbm.at[idx_v], rows_v, sem1)
      cp2 = pltpu.async_copy(gtab_hbm.at[cidx_v], crows_v, sem2)
      cp1.wait()
      cp2.wait()

      def g_body(g, carry):
        def e_body(k, svec):
          e = g * LANES + k

          def j_body(j, accs):
            r = e * CTX + j
            return tuple(accs[v] + rows_v[r, pl.ds(v * LANES, LANES)]
                         for v in range(NV))
          z = jnp.zeros((LANES,), jnp.float32)
          accs = lax.fori_loop(0, CTX, j_body, (z,) * NV)

          v1 = idx_v[pl.ds(e * CTX, LANES)]
          v2 = idx_v[pl.ds(e * CTX + 4, LANES)]
          nz = (jnp.where(v1 == 0, ones, zeros)
                + jnp.where((v2 == 0) & (lanes >= 12), ones, zeros))
          n0f = jnp.sum(nz).astype(jnp.float32)
          cnt = jnp.float32(CTX) - n0f

          t = jnp.zeros((LANES,), jnp.float32)
          for v in range(NV):
            crow = crows_v[e, pl.ds(v * LANES, LANES)]
            t = t + (accs[v] - n0f * row0[v]) * crow
          t = t / jnp.full((LANES,), cnt)
          s = jnp.sum(t)
          return jnp.where(lanes == k, jnp.full((LANES,), s), svec)

        svec = lax.fori_loop(0, LANES, e_body,
                             jnp.zeros((LANES,), jnp.float32))
        sc_v[pl.ds(g * LANES, LANES)] = svec
        return carry

      lax.fori_loop(0, CH // LANES, g_body, 0)
      pltpu.sync_copy(sc_v, out_hbm.at[pl.ds(ebase, CH)])

  return kern(ids_flat, center_ids, context_table, center_table)


def _tc_loss(scores, labels):
  s2 = scores.reshape(128, 128)
  y2 = labels.reshape(128, 128)

  def body(s_ref, y_ref, o_ref):
    s = s_ref[...]
    y = y_ref[...]
    p = jax.nn.sigmoid(s)
    loss = -(y * jnp.log(p + 1e-08) + (1.0 - y) * jnp.log(1.0 - p + 1e-08))
    o_ref[0, 0] = jnp.sum(loss) / jnp.float32(BATCH)

  out = pl.pallas_call(
      body,
      out_shape=jax.ShapeDtypeStruct((1, 1), jnp.float32),
      out_specs=pl.BlockSpec(memory_space=pltpu.SMEM),
  )(s2, y2)
  return out[0, 0]


@jax.jit
def kernel(context_ids, center_ids, labels, context_table, center_table):
  ids = context_ids.astype(jnp.int32).reshape(BATCH * CTX)
  cids = center_ids.astype(jnp.int32)
  scores = _sc_scores(ids, cids,
                      context_table.astype(jnp.float32),
                      center_table.astype(jnp.float32))
  return _tc_loss(scores, labels.astype(jnp.float32))


# speedup vs baseline: 7.8715x; 1.2769x over previous
import functools

import jax
import jax.numpy as jnp
from jax import lax
from jax.experimental import pallas as pl
from jax.experimental.pallas import tpu as pltpu
from jax.experimental.pallas import tpu_sc as plsc

VOCAB = 100000
EMBED = 64
BATCH = 16384
CTX = 20

NUM_CORES = 2
NUM_SUBCORES = 16
NW = NUM_CORES * NUM_SUBCORES
PER_W = BATCH // NW
CH = 32
N_CHUNKS = PER_W // CH
LANES = 16
NV = EMBED // LANES


def _sc_scores(ids_flat, center_ids, context_table, center_table):
  mesh = plsc.VectorSubcoreMesh(core_axis_name="c", subcore_axis_name="s")

  @functools.partial(
      pl.kernel,
      out_type=jax.ShapeDtypeStruct((BATCH,), jnp.float32),
      mesh=mesh,
      compiler_params=pltpu.CompilerParams(needs_layout_passes=False,
                                           use_tc_tiling_on_sc=False),
      scratch_types=[
          pltpu.VMEM((2, CH * CTX), jnp.int32),
          pltpu.VMEM((2, CH), jnp.int32),
          pltpu.VMEM((2, CH * CTX, EMBED), jnp.float32),
          pltpu.VMEM((2, CH, EMBED), jnp.float32),
          pltpu.VMEM((EMBED,), jnp.float32),
          pltpu.VMEM((PER_W,), jnp.float32),
          pltpu.SemaphoreType.DMA,
          pltpu.SemaphoreType.DMA,
          pltpu.SemaphoreType.DMA,
          pltpu.SemaphoreType.DMA,
      ],
  )
  def kern(ids_hbm, cids_hbm, ctab_hbm, gtab_hbm, out_hbm,
           idx_v, cidx_v, rows_v, crows_v, row0_v, sc_v,
           sem1a, sem2a, sem1b, sem2b):
    wid = lax.axis_index("s") * NUM_CORES + lax.axis_index("c")
    wbase = wid * PER_W
    lanes = lax.iota(jnp.int32, LANES)
    ones = jnp.ones((LANES,), jnp.int32)
    zeros = jnp.zeros((LANES,), jnp.int32)

    pltpu.sync_copy(ctab_hbm.at[0], row0_v)
    row0 = tuple(row0_v[pl.ds(v * LANES, LANES)] for v in range(NV))

    bufs = [(idx_v.at[0], cidx_v.at[0], rows_v.at[0], crows_v.at[0],
             sem1a, sem2a),
            (idx_v.at[1], cidx_v.at[1], rows_v.at[1], crows_v.at[1],
             sem1b, sem2b)]

    def issue(c, buf):
      idx, cidx, rows, crows, s1, s2 = buf
      ebase = wbase + c * CH
      pltpu.sync_copy(ids_hbm.at[pl.ds(ebase * CTX, CH * CTX)], idx)
      pltpu.sync_copy(cids_hbm.at[pl.ds(ebase, CH)], cidx)
      pltpu.async_copy(ctab_hbm.at[idx], rows, s1)
      pltpu.async_copy(gtab_hbm.at[cidx], crows, s2)

    def compute(c, buf):
      idx, cidx, rows, crows, s1, s2 = buf
      pltpu.make_async_copy(ctab_hbm.at[idx], rows, s1).wait()
      pltpu.make_async_copy(gtab_hbm.at[cidx], crows, s2).wait()

      def g_body(g, carry):
        def e_body(k, svec):
          e = g * LANES + k

          accs = [jnp.zeros((LANES,), jnp.float32) for _ in range(NV)]
          for j in range(CTX):
            r = e * CTX + j
            for v in range(NV):
              accs[v] = accs[v] + rows[r, pl.ds(v * LANES, LANES)]

          v1 = idx[pl.ds(e * CTX, LANES)]
          v2 = idx[pl.ds(e * CTX + 4, LANES)]
          nz = (jnp.where(v1 == 0, ones, zeros)
                + jnp.where((v2 == 0) & (lanes >= 12), ones, zeros))
          n0f = jnp.sum(nz).astype(jnp.float32)
          cnt = jnp.float32(CTX) - n0f

          t = jnp.zeros((LANES,), jnp.float32)
          for v in range(NV):
            crow = crows[e, pl.ds(v * LANES, LANES)]
            t = t + (accs[v] - n0f * row0[v]) * crow
          t = t / jnp.full((LANES,), cnt)
          s = jnp.sum(t)
          return jnp.where(lanes == k, jnp.full((LANES,), s), svec)

        svec = lax.fori_loop(0, LANES, e_body,
                             jnp.zeros((LANES,), jnp.float32))
        sc_v[pl.ds(c * CH + g * LANES, LANES)] = svec
        return carry

      lax.fori_loop(0, CH // LANES, g_body, 0)

    issue(0, bufs[0])

    def pair_body(i, carry):
      c0 = 2 * i
      issue(c0 + 1, bufs[1])
      compute(c0, bufs[0])

      @pl.when(i < N_CHUNKS // 2 - 1)
      def _():
        issue(c0 + 2, bufs[0])

      compute(c0 + 1, bufs[1])
      return carry

    lax.fori_loop(0, N_CHUNKS // 2, pair_body, 0)
    pltpu.sync_copy(sc_v, out_hbm.at[pl.ds(wbase, PER_W)])

  return kern(ids_flat, center_ids, context_table, center_table)


def _tc_loss(scores, labels):
  s2 = scores.reshape(128, 128)
  y2 = labels.reshape(128, 128)

  def body(s_ref, y_ref, o_ref):
    s = s_ref[...]
    y = y_ref[...]
    p = jax.nn.sigmoid(s)
    loss = -(y * jnp.log(p + 1e-08) + (1.0 - y) * jnp.log(1.0 - p + 1e-08))
    o_ref[0, 0] = jnp.sum(loss) / jnp.float32(BATCH)

  out = pl.pallas_call(
      body,
      out_shape=jax.ShapeDtypeStruct((1, 1), jnp.float32),
      out_specs=pl.BlockSpec(memory_space=pltpu.SMEM),
  )(s2, y2)
  return out[0, 0]


@jax.jit
def kernel(context_ids, center_ids, labels, context_table, center_table):
  ids = context_ids.astype(jnp.int32).reshape(BATCH * CTX)
  cids = center_ids.astype(jnp.int32)
  scores = _sc_scores(ids, cids,
                      context_table.astype(jnp.float32),
                      center_table.astype(jnp.float32))
  return _tc_loss(scores, labels.astype(jnp.float32))
